# baseline (speedup 1.0000x reference)
import functools

import jax
import jax.numpy as jnp
from jax import lax
from jax.experimental import pallas as pl
from jax.experimental.pallas import tpu as pltpu
from jax.experimental.pallas import tpu_sc as plsc

B, S, D, E = 4, 2048, 2048, 64
EPS = 1e-06

T = B * S
NW = 32
TPW = T // NW


def _logits_body(x_ref, scale_ref, w_ref, logits_ref):
    x = x_ref[...]
    ssq = jnp.sum(x * x, axis=1, keepdims=True)
    r = lax.rsqrt(ssq * (1.0 / D) + EPS)
    xn = x * r
    ws = w_ref[...] * scale_ref[...]
    logits_ref[...] = lax.dot_general(
        ws, xn,
        dimension_numbers=(((1,), (1,)), ((), ())),
        preferred_element_type=jnp.float32,
    )


def _route_sc_body(lg_hbm, routes_hbm, p_hbm, lg_v, routes_v, p_v):
    wid = lax.axis_index("s") * 2 + lax.axis_index("c")
    tbase = wid * TPW
    pltpu.sync_copy(lg_hbm.at[:, pl.ds(tbase, TPW)], lg_v)

    def group(g, carry):
        i0 = pl.multiple_of(g * 16, 16)
        sl = pl.ds(i0, 16)
        m = lg_v[0, sl]
        routes = jnp.zeros((16,), jnp.int32)
        for e in range(1, E):
            v = lg_v[e, sl]
            upd = v > m
            m = jnp.where(upd, v, m)
            routes = jnp.where(upd, jnp.full((16,), e, jnp.int32), routes)
        denom = jnp.zeros((16,), jnp.float32)
        for e in range(E):
            denom = denom + jnp.exp(lg_v[e, sl] - m)
        routes_v[sl] = routes
        p_v[sl] = 1.0 / denom
        return carry

    lax.fori_loop(0, TPW // 16, group, 0)
    pltpu.sync_copy(routes_v, routes_hbm.at[pl.ds(tbase, TPW)])
    pltpu.sync_copy(p_v, p_hbm.at[pl.ds(tbase, TPW)])


@functools.partial(jax.jit, static_argnames=())
def kernel(hidden_states, scale, W):
    d = hidden_states.shape[2]
    x = hidden_states.reshape(T, d)
    TB = 1024
    G = T // TB
    logits2 = pl.pallas_call(
        _logits_body,
        grid=(G,),
        in_specs=[
            pl.BlockSpec((TB, d), lambda i: (i, 0)),
            pl.BlockSpec((1, d), lambda i: (0, 0)),
            pl.BlockSpec((E, d), lambda i: (0, 0)),
        ],
        out_specs=pl.BlockSpec((E, TB), lambda i: (0, i)),
        out_shape=jax.ShapeDtypeStruct((E, T), jnp.float32),
        compiler_params=pltpu.CompilerParams(
            dimension_semantics=("parallel",),
        ),
    )(x, scale.reshape(1, d), W)

    mesh = plsc.VectorSubcoreMesh(core_axis_name="c", subcore_axis_name="s")
    routes, p = pl.kernel(
        _route_sc_body,
        mesh=mesh,
        out_type=(
            jax.ShapeDtypeStruct((T,), jnp.int32),
            jax.ShapeDtypeStruct((T,), jnp.float32),
        ),
        scratch_types=[
            pltpu.VMEM((E, TPW), jnp.float32),
            pltpu.VMEM((TPW,), jnp.int32),
            pltpu.VMEM((TPW,), jnp.float32),
        ],
    )(logits2)
    return routes, p


# speedup vs baseline: 1.6908x; 1.6908x over previous
import functools

import jax
import jax.numpy as jnp
from jax.experimental import pallas as pl
from jax.experimental.pallas import tpu as pltpu

B, S, D, E = 4, 2048, 2048, 64
EPS = 1e-06


def _router_body(x_ref, w_ref, routes_ref, p_ref):
    x = x_ref[...]
    ssq = jnp.sum(x * x, axis=1, keepdims=True)
    r = jax.lax.rsqrt(ssq * (1.0 / D) + EPS)
    xn = x * r
    logits = jax.lax.dot_general(
        w_ref[...], xn,
        dimension_numbers=(((1,), (1,)), ((), ())),
        preferred_element_type=jnp.float32,
    )
    m = jnp.max(logits, axis=0, keepdims=True)
    ids = jax.lax.broadcasted_iota(jnp.int32, logits.shape, 0)
    idx = jnp.min(jnp.where(logits == m, ids, E), axis=0)
    denom = jnp.sum(jnp.exp(logits - m), axis=0)
    routes_ref[0, 0, :] = idx
    p_ref[0, 0, :] = 1.0 / denom


@functools.partial(jax.jit, static_argnames=())
def kernel(hidden_states, scale, W):
    T = hidden_states.shape[0] * hidden_states.shape[1]
    d = hidden_states.shape[2]
    x = hidden_states.reshape(T, d)
    ws = W * scale
    TB = 1024
    G = T // TB
    routes2, p2 = pl.pallas_call(
        _router_body,
        grid=(G,),
        in_specs=[
            pl.BlockSpec((TB, d), lambda i: (i, 0)),
            pl.BlockSpec((E, d), lambda i: (0, 0)),
        ],
        out_specs=[
            pl.BlockSpec((1, 1, TB), lambda i: (i, 0, 0)),
            pl.BlockSpec((1, 1, TB), lambda i: (i, 0, 0)),
        ],
        out_shape=[
            jax.ShapeDtypeStruct((G, 1, TB), jnp.int32),
            jax.ShapeDtypeStruct((G, 1, TB), jnp.float32),
        ],
        compiler_params=pltpu.CompilerParams(
            dimension_semantics=("parallel",),
        ),
    )(x, ws)
    return routes2.reshape(T), p2.reshape(T)


# speedup vs baseline: 1.8128x; 1.0722x over previous
import functools

import jax
import jax.numpy as jnp
from jax.experimental import pallas as pl
from jax.experimental.pallas import tpu as pltpu

B, S, D, E = 4, 2048, 2048, 64
EPS = 1e-06


def _router_body(x_ref, scale_ref, w_ref, routes_ref, p_ref):
    x = x_ref[...]
    ssq = jnp.sum(x * x, axis=1, keepdims=True)
    r = jax.lax.rsqrt(ssq * (1.0 / D) + EPS)
    xn = x * r
    ws = w_ref[...] * scale_ref[...]
    logits = jax.lax.dot_general(
        ws, xn,
        dimension_numbers=(((1,), (1,)), ((), ())),
        preferred_element_type=jnp.float32,
    )
    m = jnp.max(logits, axis=0, keepdims=True)
    ids = jax.lax.broadcasted_iota(jnp.int32, logits.shape, 0)
    idx = jnp.min(jnp.where(logits == m, ids, E), axis=0)
    denom = jnp.sum(jnp.exp(logits - m), axis=0)
    routes_ref[0, 0, :] = idx
    p_ref[0, 0, :] = 1.0 / denom


@functools.partial(jax.jit, static_argnames=())
def kernel(hidden_states, scale, W):
    T = hidden_states.shape[0] * hidden_states.shape[1]
    d = hidden_states.shape[2]
    x = hidden_states.reshape(T, d)
    TB = 1024
    G = T // TB
    routes2, p2 = pl.pallas_call(
        _router_body,
        grid=(G,),
        in_specs=[
            pl.BlockSpec((TB, d), lambda i: (i, 0)),
            pl.BlockSpec((1, d), lambda i: (0, 0)),
            pl.BlockSpec((E, d), lambda i: (0, 0)),
        ],
        out_specs=[
            pl.BlockSpec((1, 1, TB), lambda i: (i, 0, 0)),
            pl.BlockSpec((1, 1, TB), lambda i: (i, 0, 0)),
        ],
        out_shape=[
            jax.ShapeDtypeStruct((G, 1, TB), jnp.int32),
            jax.ShapeDtypeStruct((G, 1, TB), jnp.float32),
        ],
        compiler_params=pltpu.CompilerParams(
            dimension_semantics=("parallel",),
        ),
    )(x, scale.reshape(1, d), W)
    return routes2.reshape(T), p2.reshape(T)


# speedup vs baseline: 1.8167x; 1.0021x over previous
import functools

import jax
import jax.numpy as jnp
from jax.experimental import pallas as pl
from jax.experimental.pallas import tpu as pltpu

B, S, D, E = 4, 2048, 2048, 64
EPS = 1e-06


def _router_body(x_ref, scale_ref, w_ref, routes_ref, p_ref):
    x = x_ref[...]
    ssq = jnp.sum(x * x, axis=1, keepdims=True)
    r = jax.lax.rsqrt(ssq * (1.0 / D) + EPS)
    xn = x * r
    ws = w_ref[...] * scale_ref[...]
    logits = jax.lax.dot_general(
        ws, xn,
        dimension_numbers=(((1,), (1,)), ((), ())),
        preferred_element_type=jnp.float32,
    )
    m = jnp.max(logits, axis=0, keepdims=True)
    ids = jax.lax.broadcasted_iota(jnp.int32, logits.shape, 0)
    idx = jnp.min(jnp.where(logits == m, ids, E), axis=0)
    denom = jnp.sum(jnp.exp(logits - m), axis=0)
    routes_ref[0, 0, :] = idx
    p_ref[0, 0, :] = 1.0 / denom


@functools.partial(jax.jit, static_argnames=())
def kernel(hidden_states, scale, W):
    T = hidden_states.shape[0] * hidden_states.shape[1]
    d = hidden_states.shape[2]
    x = hidden_states.reshape(T, d)
    TB = 2048
    G = T // TB
    routes2, p2 = pl.pallas_call(
        _router_body,
        grid=(G,),
        in_specs=[
            pl.BlockSpec((TB, d), lambda i: (i, 0)),
            pl.BlockSpec((1, d), lambda i: (0, 0)),
            pl.BlockSpec((E, d), lambda i: (0, 0)),
        ],
        out_specs=[
            pl.BlockSpec((1, 1, TB), lambda i: (i, 0, 0)),
            pl.BlockSpec((1, 1, TB), lambda i: (i, 0, 0)),
        ],
        out_shape=[
            jax.ShapeDtypeStruct((G, 1, TB), jnp.int32),
            jax.ShapeDtypeStruct((G, 1, TB), jnp.float32),
        ],
        compiler_params=pltpu.CompilerParams(
            dimension_semantics=("parallel",),
        ),
    )(x, scale.reshape(1, d), W)
    return routes2.reshape(T), p2.reshape(T)


# speedup vs baseline: 1.8480x; 1.0173x over previous
import functools

import jax
import jax.numpy as jnp
from jax.experimental import pallas as pl
from jax.experimental.pallas import tpu as pltpu

B, S, D, E = 4, 2048, 2048, 64
EPS = 1e-06


def _router_body(x_ref, scale_ref, w_ref, routes_ref, p_ref):
    x = x_ref[...]
    ssq = jnp.sum(x * x, axis=1, keepdims=True)
    r = jax.lax.rsqrt(ssq * (1.0 / D) + EPS)
    xn = x * r
    ws = w_ref[...] * scale_ref[...]
    logits = jax.lax.dot_general(
        ws, xn,
        dimension_numbers=(((1,), (1,)), ((), ())),
        preferred_element_type=jnp.float32,
    )
    m = jnp.max(logits, axis=0, keepdims=True)
    ids = jax.lax.broadcasted_iota(jnp.int32, logits.shape, 0)
    idx = jnp.min(jnp.where(logits == m, ids, E), axis=0)
    denom = jnp.sum(jnp.exp(logits - m), axis=0)
    routes_ref[0, 0, :] = idx
    p_ref[0, 0, :] = 1.0 / denom


@functools.partial(jax.jit, static_argnames=())
def kernel(hidden_states, scale, W):
    T = hidden_states.shape[0] * hidden_states.shape[1]
    d = hidden_states.shape[2]
    x = hidden_states.reshape(T, d)
    TB = 1024
    G = T // TB
    routes2, p2 = pl.pallas_call(
        _router_body,
        grid=(G,),
        in_specs=[
            pl.BlockSpec((TB, d), lambda i: (i, 0)),
            pl.BlockSpec((1, d), lambda i: (0, 0)),
            pl.BlockSpec((E, d), lambda i: (0, 0)),
        ],
        out_specs=[
            pl.BlockSpec((1, 1, TB), lambda i: (i, 0, 0)),
            pl.BlockSpec((1, 1, TB), lambda i: (i, 0, 0)),
        ],
        out_shape=[
            jax.ShapeDtypeStruct((G, 1, TB), jnp.int32),
            jax.ShapeDtypeStruct((G, 1, TB), jnp.float32),
        ],
        compiler_params=pltpu.CompilerParams(
            dimension_semantics=("parallel",),
        ),
    )(x, scale.reshape(1, d), W)
    return routes2.reshape(T), p2.reshape(T)
